# baseline (speedup 1.0000x reference)
import functools

import jax
import jax.numpy as jnp
from jax import lax
from jax.experimental import pallas as pl
from jax.experimental.pallas import tpu as pltpu
from jax.experimental.pallas import tpu_sc as plsc

B = 16
N = 512
HALF = N // 2
MAX_EDGES = N * N
THR = 0.3

NC = 2
L = 16
AL = 128

CR = 16
CHUNK = CR * N
NCHUNK_H = HALF // CR
NPAIR = NCHUNK_H // 2
S = 8192
STG = CHUNK + 2 * AL
SIZES = (8192, 4096, 2048, 1024, 512, 256, 128)
BIG = 1 << 30


def _conn_body(prior_ref, fc_ref, sc_ref, conn_ref, info_ref):
    c = (prior_ref[...] + fc_ref[0]) + sc_ref[0]
    s = (c + c.T) / 2.0
    conn_ref[0] = s
    m = (jnp.abs(s) > THR).astype(jnp.int32)

    t_total = jnp.sum(m)
    t_half = jnp.sum(m[:HALF])
    deficit = t_half & (AL - 1)

    rowcnt = jnp.sum(m, axis=1, keepdims=True).astype(jnp.float32)
    jr = lax.broadcasted_iota(jnp.int32, (N, N), 0)
    jc = lax.broadcasted_iota(jnp.int32, (N, N), 1)
    a = ((jc >= jr) & (jc < HALF)).astype(jnp.float32)
    ss = jnp.dot(a, rowcnt, preferred_element_type=jnp.float32)
    ss = ss.astype(jnp.int32)

    r_col = lax.broadcasted_iota(jnp.int32, (N, 1), 0)
    sat = (ss >= deficit) & (r_col < HALF)
    k = jnp.sum(sat.astype(jnp.int32))
    r_star = jnp.where(deficit == 0, HALF, k - 1)
    s_suffix = jnp.where(deficit == 0, 0, jnp.min(jnp.where(sat, ss, BIG)))

    lane = lax.broadcasted_iota(jnp.int32, (1, N), 1)
    info = jnp.where(lane == 0, t_total,
                     jnp.where(lane == 1, t_half,
                               jnp.where(lane == 2, r_star,
                                         jnp.where(lane == 3, s_suffix, 0))))
    info_ref[0] = info


def _tc_conn(prior, fc, sc):
    return pl.pallas_call(
        _conn_body,
        grid=(B,),
        in_specs=[
            pl.BlockSpec((N, N), lambda b: (0, 0)),
            pl.BlockSpec((1, N, N), lambda b: (b, 0, 0)),
            pl.BlockSpec((1, N, N), lambda b: (b, 0, 0)),
        ],
        out_specs=[
            pl.BlockSpec((1, N, N), lambda b: (b, 0, 0)),
            pl.BlockSpec((1, 1, N), lambda b: (b, 0, 0)),
        ],
        out_shape=[
            jax.ShapeDtypeStruct((B, N, N), jnp.float32),
            jax.ShapeDtypeStruct((B, 1, N), jnp.int32),
        ],
    )(prior, fc, sc)


def _sc_body(conn, info, out_ei, out_ea,
             db0, db1, rs0, rs1, cs0, cs1, vs0, vs1,
             totv, wrow,
             zbuf_i, zbuf_f, tail_r, tail_c, tail_v,
             sem_in0, sem_in1, sem_f0, sem_f1, sem_z):
    w = lax.axis_index("s") * NC + lax.axis_index("c")
    iota = lax.iota(jnp.int32, L)
    b = w >> 1
    h = w & 1

    def zinit(i, _):
        zbuf_i[pl.ds(i * L, L)] = jnp.zeros((L,), jnp.int32)
        zbuf_f[pl.ds(i * L, L)] = jnp.zeros((L,), jnp.float32)
        return 0
    lax.fori_loop(0, S // L, zinit, 0)

    rbase = (2 * b) * MAX_EDGES
    cbase = (2 * b + 1) * MAX_EDGES
    abase = b * MAX_EDGES

    pltpu.sync_copy(info.at[pl.ds(b * N, AL)], totv)
    iv = totv[pl.ds(0, L)]
    t_total = iv[0]
    t_half = iv[1]
    r_star = iv[2]
    s_suffix = iv[3]

    ptr0 = jnp.where(h == 0, 0, t_half)
    q0 = ptr0 & (-AL)
    deficit = ptr0 - q0

    rsA, csA, vsA = rs0, cs0, vs0
    rsB, csB, vsB = rs1, cs1, vs1

    def compact_row(r, ov, src, srow, valid, rs, cs, vs):
        rvec = jnp.full((L,), r, jnp.int32)
        for j in range(N // L):
            v = src[srow, pl.ds(j * L, L)]
            m = (jnp.abs(v) > THR) & valid
            mi = m.astype(jnp.int32)
            excl = plsc.cumsum(mi) - mi
            idx = ov + excl
            pc = plsc.all_reduce_population_count(m)
            plsc.store_scatter(vs, [idx], v, mask=m)
            plsc.store_scatter(cs, [idx], iota + (j * L), mask=m)
            plsc.store_scatter(rs, [idx], rvec, mask=m)
            ov = ov + pc
        return ov

    @pl.when(h == 1)
    def _():
        rw0 = r_star & (-8)

        def walk(t, ov):
            r8 = rw0 + 8 * t
            pltpu.sync_copy(conn.at[b, pl.ds(pl.multiple_of(r8, 8), 8)], wrow)
            for k in range(8):
                ov = compact_row(r8 + k, ov, wrow, k, r8 + k >= r_star,
                                 rsB, csB, vsB)
            return ov
        nw = (HALF - rw0) // 8
        wtot_v = lax.fori_loop(0, nw, walk, jnp.zeros((L,), jnp.int32))
        wtot = wtot_v[0]
        src0 = wtot - deficit
        rvals, cvals, vvals = [], [], []
        for k in range(AL // L):
            idx = src0 + (k * L) + iota
            idx = jnp.maximum(idx, 0)
            rvals.append(plsc.load_gather(rsB, [idx]))
            cvals.append(plsc.load_gather(csB, [idx]))
            vvals.append(plsc.load_gather(vsB, [idx]))
        for k in range(AL // L):
            rsB[pl.ds(k * L, L)] = rvals[k]
            csB[pl.ds(k * L, L)] = cvals[k]
            vsB[pl.ds(k * L, L)] = vvals[k]

        zstart = (t_total + (AL - 1)) & (-AL)
        zlen = MAX_EDGES - zstart
        n_full = zlen // S
        rem = zlen - n_full * S

        def zfill(i, _):
            d = zstart + i * S
            pltpu.async_copy(
                zbuf_i, out_ei.at[pl.ds(pl.multiple_of(rbase + d, AL), S)],
                sem_z)
            pltpu.async_copy(
                zbuf_i, out_ei.at[pl.ds(pl.multiple_of(cbase + d, AL), S)],
                sem_z)
            pltpu.async_copy(
                zbuf_f, out_ea.at[pl.ds(pl.multiple_of(abase + d, AL), S)],
                sem_z)
            return 0
        lax.fori_loop(0, n_full, zfill, 0)

        zoff = jnp.int32(0)
        for sz in SIZES[1:]:
            zpred = (rem & sz) != 0

            @pl.when(zpred)
            def _(zoff=zoff, sz=sz):
                d = zstart + n_full * S + zoff
                pltpu.async_copy(
                    zbuf_i.at[pl.ds(0, sz)],
                    out_ei.at[pl.ds(pl.multiple_of(rbase + d, AL), sz)],
                    sem_z)
                pltpu.async_copy(
                    zbuf_i.at[pl.ds(0, sz)],
                    out_ei.at[pl.ds(pl.multiple_of(cbase + d, AL), sz)],
                    sem_z)
                pltpu.async_copy(
                    zbuf_f.at[pl.ds(0, sz)],
                    out_ea.at[pl.ds(pl.multiple_of(abase + d, AL), sz)],
                    sem_z)
            zoff = jnp.where(zpred, zoff + sz, zoff)

    cbase0 = jnp.where(h == 0, 0, NCHUNK_H)

    def chunk_src(ci):
        return conn.at[b, pl.ds(pl.multiple_of(ci * CR, 8), CR)]

    pltpu.async_copy(chunk_src(cbase0), db0, sem_in0)
    pltpu.async_copy(chunk_src(cbase0 + 1), db1, sem_in1)

    true_v = jnp.ones((L,), jnp.bool_)

    def compact_chunk(db, ci, o0, rs, cs, vs):
        def row_body(i, ov):
            return compact_row(ci * CR + i, ov, db, i, true_v, rs, cs, vs)
        return lax.fori_loop(0, CR, row_body, o0)

    def flush(fl, q, rs, cs, vs, sem):
        off = jnp.int32(0)
        for sz in SIZES:
            pred = (fl & sz) != 0

            @pl.when(pred)
            def _(off=off, sz=sz):
                so = pl.multiple_of(off, AL)
                d = q + off
                pltpu.async_copy(
                    rs.at[pl.ds(so, sz)],
                    out_ei.at[pl.ds(pl.multiple_of(rbase + d, AL), sz)], sem)
                pltpu.async_copy(
                    cs.at[pl.ds(so, sz)],
                    out_ei.at[pl.ds(pl.multiple_of(cbase + d, AL), sz)], sem)
                pltpu.async_copy(
                    vs.at[pl.ds(so, sz)],
                    out_ea.at[pl.ds(pl.multiple_of(abase + d, AL), sz)], sem)
            off = jnp.where(pred, off + sz, off)

    def drain_flush(fl, rs, cs, vs, sem):
        for sz in SIZES:
            pred = (fl & sz) != 0

            @pl.when(pred)
            def _(sz=sz):
                pltpu.make_async_copy(out_ei.at[pl.ds(0, sz)],
                                      rs.at[pl.ds(0, sz)], sem).wait()
                pltpu.make_async_copy(out_ei.at[pl.ds(0, sz)],
                                      cs.at[pl.ds(0, sz)], sem).wait()
                pltpu.make_async_copy(out_ea.at[pl.ds(0, sz)],
                                      vs.at[pl.ds(0, sz)], sem).wait()

    def carry_tail(fl_src, rs_s, cs_s, vs_s, rs_d, cs_d, vs_d):
        for k in range(AL // L):
            sp = pl.ds(pl.multiple_of(fl_src + k * L, L), L)
            rs_d[pl.ds(k * L, L)] = rs_s[sp]
            cs_d[pl.ds(k * L, L)] = cs_s[sp]
            vs_d[pl.ds(k * L, L)] = vs_s[sp]

    def pair_body(p, carry):
        ptr, q, flA, flB = carry
        cA = cbase0 + 2 * p
        cB = cA + 1

        pltpu.make_async_copy(chunk_src(cbase0), db0, sem_in0).wait()
        drain_flush(flA, rsA, csA, vsA, sem_f0)
        carry_tail(flB, rsB, csB, vsB, rsA, csA, vsA)
        ov = compact_chunk(db0, cA, jnp.full((L,), ptr - q, jnp.int32),
                           rsA, csA, vsA)
        o = ov[0]
        flA = o & (-AL)
        flush(flA, q, rsA, csA, vsA, sem_f0)
        ptr = q + o
        q = q + flA

        @pl.when(2 * p + 2 < NCHUNK_H)
        def _():
            pltpu.async_copy(chunk_src(cA + 2), db0, sem_in0)

        pltpu.make_async_copy(chunk_src(cbase0), db1, sem_in1).wait()
        drain_flush(flB, rsB, csB, vsB, sem_f1)
        carry_tail(flA, rsA, csA, vsA, rsB, csB, vsB)
        ov = compact_chunk(db1, cB, jnp.full((L,), ptr - q, jnp.int32),
                           rsB, csB, vsB)
        o = ov[0]
        flB = o & (-AL)
        flush(flB, q, rsB, csB, vsB, sem_f1)
        ptr = q + o
        q = q + flB

        @pl.when(2 * p + 3 < NCHUNK_H)
        def _():
            pltpu.async_copy(chunk_src(cB + 2), db1, sem_in1)

        return ptr, q, flA, flB

    ptr, q, flA, flB = lax.fori_loop(
        0, NPAIR, pair_body, (ptr0, q0, jnp.int32(0), jnp.int32(0)))

    drain_flush(flA, rsA, csA, vsA, sem_f0)
    drain_flush(flB, rsB, csB, vsB, sem_f1)

    tail = ptr - q

    @pl.when((h == 1) & (tail > 0))
    def _():
        for k in range(AL // L):
            keep = (iota + k * L) < tail
            src = pl.ds(pl.multiple_of(flB + k * L, L), L)
            tail_r[pl.ds(k * L, L)] = jnp.where(keep, rsB[src], 0)
            tail_c[pl.ds(k * L, L)] = jnp.where(keep, csB[src], 0)
            tail_v[pl.ds(k * L, L)] = jnp.where(keep, vsB[src], 0.0)
        pltpu.sync_copy(tail_r,
                        out_ei.at[pl.ds(pl.multiple_of(rbase + q, AL), AL)])
        pltpu.sync_copy(tail_c,
                        out_ei.at[pl.ds(pl.multiple_of(cbase + q, AL), AL)])
        pltpu.sync_copy(tail_v,
                        out_ea.at[pl.ds(pl.multiple_of(abase + q, AL), AL)])

    @pl.when(h == 1)
    def _():
        zstart = (t_total + (AL - 1)) & (-AL)
        zlen = MAX_EDGES - zstart
        n_full = zlen // S
        rem = zlen - n_full * S

        def zdrain(i, _):
            pltpu.make_async_copy(out_ei.at[pl.ds(0, S)], zbuf_i,
                                  sem_z).wait()
            pltpu.make_async_copy(out_ei.at[pl.ds(0, S)], zbuf_i,
                                  sem_z).wait()
            pltpu.make_async_copy(out_ea.at[pl.ds(0, S)], zbuf_f,
                                  sem_z).wait()
            return 0
        lax.fori_loop(0, n_full, zdrain, 0)
        for sz in SIZES[1:]:
            zpred = (rem & sz) != 0

            @pl.when(zpred)
            def _(sz=sz):
                pltpu.make_async_copy(out_ei.at[pl.ds(0, sz)],
                                      zbuf_i.at[pl.ds(0, sz)], sem_z).wait()
                pltpu.make_async_copy(out_ei.at[pl.ds(0, sz)],
                                      zbuf_i.at[pl.ds(0, sz)], sem_z).wait()
                pltpu.make_async_copy(out_ea.at[pl.ds(0, sz)],
                                      zbuf_f.at[pl.ds(0, sz)], sem_z).wait()


def _sc_compact(conn, info):
    mesh = plsc.VectorSubcoreMesh(core_axis_name="c", subcore_axis_name="s")
    f = functools.partial(
        pl.kernel,
        out_type=[
            jax.ShapeDtypeStruct((B * 2 * MAX_EDGES,), jnp.int32),
            jax.ShapeDtypeStruct((B * MAX_EDGES,), jnp.float32),
        ],
        mesh=mesh,
        compiler_params=pltpu.CompilerParams(needs_layout_passes=False),
        scratch_types=[
            pltpu.VMEM((CR, N), jnp.float32),
            pltpu.VMEM((CR, N), jnp.float32),
            pltpu.VMEM((STG,), jnp.int32),
            pltpu.VMEM((STG,), jnp.int32),
            pltpu.VMEM((STG,), jnp.int32),
            pltpu.VMEM((STG,), jnp.int32),
            pltpu.VMEM((STG,), jnp.float32),
            pltpu.VMEM((STG,), jnp.float32),
            pltpu.VMEM((AL,), jnp.int32),
            pltpu.VMEM((8, N), jnp.float32),
            pltpu.VMEM((S,), jnp.int32),
            pltpu.VMEM((S,), jnp.float32),
            pltpu.VMEM((AL,), jnp.int32),
            pltpu.VMEM((AL,), jnp.int32),
            pltpu.VMEM((AL,), jnp.float32),
            pltpu.SemaphoreType.DMA,
            pltpu.SemaphoreType.DMA,
            pltpu.SemaphoreType.DMA,
            pltpu.SemaphoreType.DMA,
            pltpu.SemaphoreType.DMA,
        ],
    )(_sc_body)
    return f(conn, info)


def kernel(functional_connectivity, structural_connectivity,
           connectivity_prior):
    conn, info = _tc_conn(connectivity_prior, functional_connectivity,
                          structural_connectivity)
    ei_flat, ea_flat = _sc_compact(conn, info.reshape(B * N))
    edge_index = ei_flat.reshape(B, 2, MAX_EDGES)
    edge_attr = ea_flat.reshape(B, MAX_EDGES)
    return edge_index, edge_attr


# speedup vs baseline: 1.1512x; 1.1512x over previous
import functools

import jax
import jax.numpy as jnp
from jax import lax
from jax.experimental import pallas as pl
from jax.experimental.pallas import tpu as pltpu
from jax.experimental.pallas import tpu_sc as plsc

B = 16
N = 512
HALF = N // 2
MAX_EDGES = N * N
THR = 0.3

NC = 2
L = 16
AL = 128

CR = 16
CHUNK = CR * N
NCHUNK_H = HALF // CR
NPAIR = NCHUNK_H // 2
S = 8192
STG = CHUNK + 2 * AL
SIZES = (8192, 4096, 2048, 1024, 512, 256, 128)
BIG = 1 << 30


def _conn_body(prior_ref, fc_ref, sc_ref, conn_ref, info_ref):
    c = (prior_ref[...] + fc_ref[0]) + sc_ref[0]
    s = (c + c.T) / 2.0
    conn_ref[0] = s
    m = (jnp.abs(s) > THR).astype(jnp.int32)

    t_total = jnp.sum(m)
    t_half = jnp.sum(m[:HALF])
    deficit = t_half & (AL - 1)

    rowcnt = jnp.sum(m, axis=1, keepdims=True).astype(jnp.float32)
    jr = lax.broadcasted_iota(jnp.int32, (N, N), 0)
    jc = lax.broadcasted_iota(jnp.int32, (N, N), 1)
    a = ((jc >= jr) & (jc < HALF)).astype(jnp.float32)
    ss = jnp.dot(a, rowcnt, preferred_element_type=jnp.float32)
    ss = ss.astype(jnp.int32)

    r_col = lax.broadcasted_iota(jnp.int32, (N, 1), 0)
    sat = (ss >= deficit) & (r_col < HALF)
    k = jnp.sum(sat.astype(jnp.int32))
    r_star = jnp.where(deficit == 0, HALF, k - 1)
    s_suffix = jnp.where(deficit == 0, 0, jnp.min(jnp.where(sat, ss, BIG)))

    lane = lax.broadcasted_iota(jnp.int32, (1, N), 1)
    info = jnp.where(lane == 0, t_total,
                     jnp.where(lane == 1, t_half,
                               jnp.where(lane == 2, r_star,
                                         jnp.where(lane == 3, s_suffix, 0))))
    info_ref[0] = info


def _tc_conn(prior, fc, sc):
    return pl.pallas_call(
        _conn_body,
        grid=(B,),
        in_specs=[
            pl.BlockSpec((N, N), lambda b: (0, 0)),
            pl.BlockSpec((1, N, N), lambda b: (b, 0, 0)),
            pl.BlockSpec((1, N, N), lambda b: (b, 0, 0)),
        ],
        out_specs=[
            pl.BlockSpec((1, N, N), lambda b: (b, 0, 0)),
            pl.BlockSpec((1, 1, N), lambda b: (b, 0, 0)),
        ],
        out_shape=[
            jax.ShapeDtypeStruct((B, N, N), jnp.float32),
            jax.ShapeDtypeStruct((B, 1, N), jnp.int32),
        ],
    )(prior, fc, sc)


def _sc_body(conn, info, out_ei, out_ea,
             db0, db1, rs0, rs1, cs0, cs1, vs0, vs1,
             totv, wrow,
             zbuf_i, zbuf_f, tail_r, tail_c, tail_v,
             sem_in0, sem_in1, sem_f0, sem_f1, sem_z):
    w = lax.axis_index("s") * NC + lax.axis_index("c")
    iota = lax.iota(jnp.int32, L)
    b = w >> 1
    h = w & 1

    def zinit(i, _):
        zbuf_i[pl.ds(i * L, L)] = jnp.zeros((L,), jnp.int32)
        zbuf_f[pl.ds(i * L, L)] = jnp.zeros((L,), jnp.float32)
        return 0
    lax.fori_loop(0, S // L, zinit, 0)

    rbase = (2 * b) * MAX_EDGES
    cbase = (2 * b + 1) * MAX_EDGES
    abase = b * MAX_EDGES

    pltpu.sync_copy(info.at[pl.ds(b * N, AL)], totv)
    iv = totv[pl.ds(0, L)]
    t_total = iv[0]
    t_half = iv[1]
    r_star = iv[2]
    s_suffix = iv[3]

    ptr0 = jnp.where(h == 0, 0, t_half)
    q0 = ptr0 & (-AL)
    deficit = ptr0 - q0

    rsA, csA, vsA = rs0, cs0, vs0
    rsB, csB, vsB = rs1, cs1, vs1

    def compact_row(r, o, src, srow, valid, rs, cs, vs):
        rvec = jnp.full((L,), r, jnp.int32)
        for j in range(N // L):
            v = src[srow, pl.ds(j * L, L)]
            m = (jnp.abs(v) > THR) & valid
            cnt = plsc.all_reduce_population_count(m)[0]
            plsc.store_compressed(vs.at[pl.ds(o, L)], v, mask=m)
            plsc.store_compressed(cs.at[pl.ds(o, L)], iota + (j * L), mask=m)
            plsc.store_compressed(rs.at[pl.ds(o, L)], rvec, mask=m)
            o = o + cnt
        return o

    @pl.when(h == 1)
    def _():
        rw0 = r_star & (-8)

        def walk(t, o):
            r8 = rw0 + 8 * t
            pltpu.sync_copy(conn.at[b, pl.ds(pl.multiple_of(r8, 8), 8)], wrow)
            for k in range(8):
                o = compact_row(r8 + k, o, wrow, k, r8 + k >= r_star,
                                rsB, csB, vsB)
            return o
        nw = (HALF - rw0) // 8
        wtot = lax.fori_loop(0, nw, walk, jnp.int32(0))
        src0 = wtot - deficit
        rvals, cvals, vvals = [], [], []
        for k in range(AL // L):
            idx = src0 + (k * L) + iota
            idx = jnp.maximum(idx, 0)
            rvals.append(plsc.load_gather(rsB, [idx]))
            cvals.append(plsc.load_gather(csB, [idx]))
            vvals.append(plsc.load_gather(vsB, [idx]))
        for k in range(AL // L):
            rsB[pl.ds(k * L, L)] = rvals[k]
            csB[pl.ds(k * L, L)] = cvals[k]
            vsB[pl.ds(k * L, L)] = vvals[k]

        zstart = (t_total + (AL - 1)) & (-AL)
        zlen = MAX_EDGES - zstart
        n_full = zlen // S
        rem = zlen - n_full * S

        def zfill(i, _):
            d = zstart + i * S
            pltpu.async_copy(
                zbuf_i, out_ei.at[pl.ds(pl.multiple_of(rbase + d, AL), S)],
                sem_z)
            pltpu.async_copy(
                zbuf_i, out_ei.at[pl.ds(pl.multiple_of(cbase + d, AL), S)],
                sem_z)
            pltpu.async_copy(
                zbuf_f, out_ea.at[pl.ds(pl.multiple_of(abase + d, AL), S)],
                sem_z)
            return 0
        lax.fori_loop(0, n_full, zfill, 0)

        zoff = jnp.int32(0)
        for sz in SIZES[1:]:
            zpred = (rem & sz) != 0

            @pl.when(zpred)
            def _(zoff=zoff, sz=sz):
                d = zstart + n_full * S + zoff
                pltpu.async_copy(
                    zbuf_i.at[pl.ds(0, sz)],
                    out_ei.at[pl.ds(pl.multiple_of(rbase + d, AL), sz)],
                    sem_z)
                pltpu.async_copy(
                    zbuf_i.at[pl.ds(0, sz)],
                    out_ei.at[pl.ds(pl.multiple_of(cbase + d, AL), sz)],
                    sem_z)
                pltpu.async_copy(
                    zbuf_f.at[pl.ds(0, sz)],
                    out_ea.at[pl.ds(pl.multiple_of(abase + d, AL), sz)],
                    sem_z)
            zoff = jnp.where(zpred, zoff + sz, zoff)

    cbase0 = jnp.where(h == 0, 0, NCHUNK_H)

    def chunk_src(ci):
        return conn.at[b, pl.ds(pl.multiple_of(ci * CR, 8), CR)]

    pltpu.async_copy(chunk_src(cbase0), db0, sem_in0)
    pltpu.async_copy(chunk_src(cbase0 + 1), db1, sem_in1)

    true_v = jnp.ones((L,), jnp.bool_)

    def compact_chunk(db, ci, o0, rs, cs, vs):
        def row_body(i, o):
            return compact_row(ci * CR + i, o, db, i, true_v, rs, cs, vs)
        return lax.fori_loop(0, CR, row_body, o0)

    def flush(fl, q, rs, cs, vs, sem):
        off = jnp.int32(0)
        for sz in SIZES:
            pred = (fl & sz) != 0

            @pl.when(pred)
            def _(off=off, sz=sz):
                so = pl.multiple_of(off, AL)
                d = q + off
                pltpu.async_copy(
                    rs.at[pl.ds(so, sz)],
                    out_ei.at[pl.ds(pl.multiple_of(rbase + d, AL), sz)], sem)
                pltpu.async_copy(
                    cs.at[pl.ds(so, sz)],
                    out_ei.at[pl.ds(pl.multiple_of(cbase + d, AL), sz)], sem)
                pltpu.async_copy(
                    vs.at[pl.ds(so, sz)],
                    out_ea.at[pl.ds(pl.multiple_of(abase + d, AL), sz)], sem)
            off = jnp.where(pred, off + sz, off)

    def drain_flush(fl, rs, cs, vs, sem):
        for sz in SIZES:
            pred = (fl & sz) != 0

            @pl.when(pred)
            def _(sz=sz):
                pltpu.make_async_copy(out_ei.at[pl.ds(0, sz)],
                                      rs.at[pl.ds(0, sz)], sem).wait()
                pltpu.make_async_copy(out_ei.at[pl.ds(0, sz)],
                                      cs.at[pl.ds(0, sz)], sem).wait()
                pltpu.make_async_copy(out_ea.at[pl.ds(0, sz)],
                                      vs.at[pl.ds(0, sz)], sem).wait()

    def carry_tail(fl_src, rs_s, cs_s, vs_s, rs_d, cs_d, vs_d):
        for k in range(AL // L):
            sp = pl.ds(pl.multiple_of(fl_src + k * L, L), L)
            rs_d[pl.ds(k * L, L)] = rs_s[sp]
            cs_d[pl.ds(k * L, L)] = cs_s[sp]
            vs_d[pl.ds(k * L, L)] = vs_s[sp]

    def pair_body(p, carry):
        ptr, q, flA, flB = carry
        cA = cbase0 + 2 * p
        cB = cA + 1

        pltpu.make_async_copy(chunk_src(cbase0), db0, sem_in0).wait()
        drain_flush(flA, rsA, csA, vsA, sem_f0)
        carry_tail(flB, rsB, csB, vsB, rsA, csA, vsA)
        o = compact_chunk(db0, cA, ptr - q, rsA, csA, vsA)
        flA = o & (-AL)
        flush(flA, q, rsA, csA, vsA, sem_f0)
        ptr = q + o
        q = q + flA

        @pl.when(2 * p + 2 < NCHUNK_H)
        def _():
            pltpu.async_copy(chunk_src(cA + 2), db0, sem_in0)

        pltpu.make_async_copy(chunk_src(cbase0), db1, sem_in1).wait()
        drain_flush(flB, rsB, csB, vsB, sem_f1)
        carry_tail(flA, rsA, csA, vsA, rsB, csB, vsB)
        o = compact_chunk(db1, cB, ptr - q, rsB, csB, vsB)
        flB = o & (-AL)
        flush(flB, q, rsB, csB, vsB, sem_f1)
        ptr = q + o
        q = q + flB

        @pl.when(2 * p + 3 < NCHUNK_H)
        def _():
            pltpu.async_copy(chunk_src(cB + 2), db1, sem_in1)

        return ptr, q, flA, flB

    ptr, q, flA, flB = lax.fori_loop(
        0, NPAIR, pair_body, (ptr0, q0, jnp.int32(0), jnp.int32(0)))

    drain_flush(flA, rsA, csA, vsA, sem_f0)
    drain_flush(flB, rsB, csB, vsB, sem_f1)

    tail = ptr - q

    @pl.when((h == 1) & (tail > 0))
    def _():
        for k in range(AL // L):
            keep = (iota + k * L) < tail
            src = pl.ds(pl.multiple_of(flB + k * L, L), L)
            tail_r[pl.ds(k * L, L)] = jnp.where(keep, rsB[src], 0)
            tail_c[pl.ds(k * L, L)] = jnp.where(keep, csB[src], 0)
            tail_v[pl.ds(k * L, L)] = jnp.where(keep, vsB[src], 0.0)
        pltpu.sync_copy(tail_r,
                        out_ei.at[pl.ds(pl.multiple_of(rbase + q, AL), AL)])
        pltpu.sync_copy(tail_c,
                        out_ei.at[pl.ds(pl.multiple_of(cbase + q, AL), AL)])
        pltpu.sync_copy(tail_v,
                        out_ea.at[pl.ds(pl.multiple_of(abase + q, AL), AL)])

    @pl.when(h == 1)
    def _():
        zstart = (t_total + (AL - 1)) & (-AL)
        zlen = MAX_EDGES - zstart
        n_full = zlen // S
        rem = zlen - n_full * S

        def zdrain(i, _):
            pltpu.make_async_copy(out_ei.at[pl.ds(0, S)], zbuf_i,
                                  sem_z).wait()
            pltpu.make_async_copy(out_ei.at[pl.ds(0, S)], zbuf_i,
                                  sem_z).wait()
            pltpu.make_async_copy(out_ea.at[pl.ds(0, S)], zbuf_f,
                                  sem_z).wait()
            return 0
        lax.fori_loop(0, n_full, zdrain, 0)
        for sz in SIZES[1:]:
            zpred = (rem & sz) != 0

            @pl.when(zpred)
            def _(sz=sz):
                pltpu.make_async_copy(out_ei.at[pl.ds(0, sz)],
                                      zbuf_i.at[pl.ds(0, sz)], sem_z).wait()
                pltpu.make_async_copy(out_ei.at[pl.ds(0, sz)],
                                      zbuf_i.at[pl.ds(0, sz)], sem_z).wait()
                pltpu.make_async_copy(out_ea.at[pl.ds(0, sz)],
                                      zbuf_f.at[pl.ds(0, sz)], sem_z).wait()


def _sc_compact(conn, info):
    mesh = plsc.VectorSubcoreMesh(core_axis_name="c", subcore_axis_name="s")
    f = functools.partial(
        pl.kernel,
        out_type=[
            jax.ShapeDtypeStruct((B * 2 * MAX_EDGES,), jnp.int32),
            jax.ShapeDtypeStruct((B * MAX_EDGES,), jnp.float32),
        ],
        mesh=mesh,
        compiler_params=pltpu.CompilerParams(needs_layout_passes=False),
        scratch_types=[
            pltpu.VMEM((CR, N), jnp.float32),
            pltpu.VMEM((CR, N), jnp.float32),
            pltpu.VMEM((STG,), jnp.int32),
            pltpu.VMEM((STG,), jnp.int32),
            pltpu.VMEM((STG,), jnp.int32),
            pltpu.VMEM((STG,), jnp.int32),
            pltpu.VMEM((STG,), jnp.float32),
            pltpu.VMEM((STG,), jnp.float32),
            pltpu.VMEM((AL,), jnp.int32),
            pltpu.VMEM((8, N), jnp.float32),
            pltpu.VMEM((S,), jnp.int32),
            pltpu.VMEM((S,), jnp.float32),
            pltpu.VMEM((AL,), jnp.int32),
            pltpu.VMEM((AL,), jnp.int32),
            pltpu.VMEM((AL,), jnp.float32),
            pltpu.SemaphoreType.DMA,
            pltpu.SemaphoreType.DMA,
            pltpu.SemaphoreType.DMA,
            pltpu.SemaphoreType.DMA,
            pltpu.SemaphoreType.DMA,
        ],
    )(_sc_body)
    return f(conn, info)


def kernel(functional_connectivity, structural_connectivity,
           connectivity_prior):
    conn, info = _tc_conn(connectivity_prior, functional_connectivity,
                          structural_connectivity)
    ei_flat, ea_flat = _sc_compact(conn, info.reshape(B * N))
    edge_index = ei_flat.reshape(B, 2, MAX_EDGES)
    edge_attr = ea_flat.reshape(B, MAX_EDGES)
    return edge_index, edge_attr
